# Initial kernel scaffold; baseline (speedup 1.0000x reference)
import jax
import jax.numpy as jnp
from jax.experimental import pallas as pl


def kernel(data, labels):
    raise NotImplementedError("write your pallas kernel here")


# speedup vs baseline: 1656.6259x; 1656.6259x over previous
import jax
import jax.numpy as jnp
from jax.experimental import pallas as pl


def _loss_body(x_ref, lab_col_ref, lab_row_ref, out_ref):
    x = x_ref[...]
    B = x.shape[0]
    g = jax.lax.dot_general(
        x, x, (((1,), (1,)), ((), ())), preferred_element_type=jnp.float32
    )
    sq = jnp.sum(x * x, axis=1, keepdims=True)
    d2 = sq + sq.reshape(1, B) - 2.0 * g
    d2 = jnp.maximum(d2, 0.0)
    dist = jnp.sqrt(d2)

    same = lab_col_ref[...] == lab_row_ref[...]
    row_i = jax.lax.broadcasted_iota(jnp.int32, (B, B), 0)
    col_j = jax.lax.broadcasted_iota(jnp.int32, (B, B), 1)
    triu = row_i < col_j

    zero = jnp.zeros_like(d2)
    sum_same = jnp.sum(jnp.where(triu & same, d2, zero))
    sum_opp = jnp.sum(jnp.where(triu & (~same), (1.0 - dist) ** 2, zero))
    n_same = jnp.sum(jnp.where(triu & same, jnp.ones_like(d2), zero))
    n_total = jnp.float32(B * (B - 1) // 2)
    final = sum_same / n_same + sum_opp / (n_total - n_same)
    out_ref[...] = final.reshape(1, 1)


def kernel(data, labels):
    B = data.shape[0]
    lab_col = labels.reshape(B, 1)
    lab_row = labels.reshape(1, B)
    out = pl.pallas_call(
        _loss_body,
        out_shape=jax.ShapeDtypeStruct((1, 1), jnp.float32),
    )(data, lab_col, lab_row)
    return out[0, 0]


# speedup vs baseline: 2019.0538x; 1.2188x over previous
import jax
import jax.numpy as jnp
from jax.experimental import pallas as pl


def _loss_body(x_ref, lab_ref, out_ref):
    x = x_ref[...]
    B, d = x.shape
    s = lab_ref[...].astype(jnp.float32)
    t = 1.0 - s

    xx = x * x
    n_row = jax.lax.dot_general(
        jnp.ones((1, d), jnp.float32), xx, (((1,), (1,)), ((), ())),
        preferred_element_type=jnp.float32,
    )
    n_col = jnp.sum(xx, axis=1, keepdims=True)

    g = jax.lax.dot_general(
        x, x, (((1,), (1,)), ((), ())), preferred_element_type=jnp.float32
    )

    sum_s = jnp.sum(s)
    sum_t = jnp.float32(B) - sum_s
    sn = jnp.sum(s * n_row)
    tn = jnp.sum(t * n_row)
    xs = jax.lax.dot_general(
        s, x, (((1,), (0,)), ((), ())), preferred_element_type=jnp.float32
    )
    xt = jax.lax.dot_general(
        t, x, (((1,), (0,)), ((), ())), preferred_element_type=jnp.float32
    )
    ss = jnp.sum(xs * xs)
    tt = jnp.sum(xt * xt)
    st = jnp.sum(xs * xt)

    sum_same_d2_full = 2.0 * (sn * sum_s - ss) + 2.0 * (tn * sum_t - tt)
    sum_opp_d2_full = 2.0 * (sn * sum_t + tn * sum_s - 2.0 * st)

    d2 = jnp.maximum(n_col + n_row - 2.0 * g, 0.0)
    dist = jnp.sqrt(d2)
    row_w = jnp.sum(dist * t, axis=1, keepdims=True)
    sum_opp_d_full = 2.0 * jax.lax.dot_general(
        s, row_w, (((1,), (0,)), ((), ())), preferred_element_type=jnp.float32
    )[0, 0]

    n_same = 0.5 * (sum_s * sum_s - sum_s + sum_t * sum_t - sum_t)
    n_opp = sum_s * sum_t
    mean_same = (0.5 * sum_same_d2_full) / n_same
    mean_opp = (n_opp - sum_opp_d_full + 0.5 * sum_opp_d2_full) / n_opp
    out_ref[...] = (mean_same + mean_opp).reshape(1, 1)


def kernel(data, labels):
    B = data.shape[0]
    out = pl.pallas_call(
        _loss_body,
        out_shape=jax.ShapeDtypeStruct((1, 1), jnp.float32),
    )(data, labels.reshape(1, B))
    return out[0, 0]


# speedup vs baseline: 2214.8167x; 1.0970x over previous
import jax
import jax.numpy as jnp
from jax.experimental import pallas as pl


def _loss_body(x_ref, lab_ref, out_ref):
    x = x_ref[...]
    B, d = x.shape
    s = lab_ref[...].astype(jnp.float32)
    t = 1.0 - s

    xx = x * x
    n_row = jax.lax.dot_general(
        jnp.ones((1, d), jnp.float32), xx, (((1,), (1,)), ((), ())),
        preferred_element_type=jnp.float32,
    )
    n_col = jnp.sum(xx, axis=1, keepdims=True)

    g = jax.lax.dot_general(
        x, x, (((1,), (1,)), ((), ())), preferred_element_type=jnp.float32
    )

    sum_s = jnp.sum(s)
    sum_t = jnp.float32(B) - sum_s
    sn = jnp.sum(s * n_row)
    tn = jnp.sum(t * n_row)
    xs = jax.lax.dot_general(
        s, x, (((1,), (0,)), ((), ())), preferred_element_type=jnp.float32
    )
    xt = jax.lax.dot_general(
        t, x, (((1,), (0,)), ((), ())), preferred_element_type=jnp.float32
    )
    ss = jnp.sum(xs * xs)
    tt = jnp.sum(xt * xt)
    st = jnp.sum(xs * xt)

    sum_same_d2_full = 2.0 * (sn * sum_s - ss) + 2.0 * (tn * sum_t - tt)
    sum_opp_d2_full = 2.0 * (sn * sum_t + tn * sum_s - 2.0 * st)

    w = jnp.maximum((n_col + n_row - 2.0 * g) * t, 1e-30)
    dist_w = w * jax.lax.rsqrt(w)
    row_w = jnp.sum(dist_w, axis=1, keepdims=True)
    sum_opp_d_full = 2.0 * jax.lax.dot_general(
        s, row_w, (((1,), (0,)), ((), ())), preferred_element_type=jnp.float32
    )[0, 0]

    n_same = 0.5 * (sum_s * sum_s - sum_s + sum_t * sum_t - sum_t)
    n_opp = sum_s * sum_t
    mean_same = (0.5 * sum_same_d2_full) / n_same
    mean_opp = (n_opp - sum_opp_d_full + 0.5 * sum_opp_d2_full) / n_opp
    out_ref[...] = (mean_same + mean_opp).reshape(1, 1)


def kernel(data, labels):
    B = data.shape[0]
    out = pl.pallas_call(
        _loss_body,
        out_shape=jax.ShapeDtypeStruct((1, 1), jnp.float32),
    )(data, labels.reshape(1, B))
    return out[0, 0]
